# baseline (speedup 1.0000x reference)
import functools

import numpy as np
from itertools import combinations

import jax
import jax.numpy as jnp
from jax import lax
from jax.experimental import pallas as pl
from jax.experimental.pallas import tpu as pltpu
from jax.experimental.pallas import tpu_sc as plsc

F = 26
E = 16
FP = 32
FE = F * E
FEP = FP * E
B = 4096
V = 1000000

_PAIRS = list(combinations(range(F), 2))
_LEFT = np.array([p[0] for p in _PAIRS], dtype=np.int32)
_RIGHT = np.array([p[1] for p in _PAIRS], dtype=np.int32)
_NP = len(_PAIRS)

_PSCAT = np.zeros((_NP, F * F), np.float32)
_PSCAT[np.arange(_NP), _LEFT * F + _RIGHT] = 1.0
_EYE_E = np.eye(E, dtype=np.float32)
_Q0 = np.einsum('ij,ef->iejf', np.triu(np.ones((F, F), np.float32), 1),
                _EYE_E).reshape(FE, FE)


_CB = 8192
_CBR = _CB // 8


def _repack_body(x_ref, i_ref, o_ref):
    z = lax.dot_general(x_ref[...], i_ref[...], (((0,), (0,)), ((), ())),
                        preferred_element_type=jnp.float32)
    z3 = z.reshape(_CBR, 8, 16)
    o_ref[...] = jnp.concatenate([z3[:, fo, :] for fo in range(8)], axis=1)


def _repack(vt, i16):
    nb = (V + _CB - 1) // _CB
    return pl.pallas_call(
        _repack_body,
        grid=(nb,),
        out_shape=jax.ShapeDtypeStruct((V // 8, 128), jnp.float32),
        in_specs=[pl.BlockSpec((E, _CB), lambda i: (0, i)),
                  pl.BlockSpec((E, E), lambda i: (0, 0))],
        out_specs=pl.BlockSpec((_CBR, 128), lambda i: (i, 0)),
    )(vt, i16)


_NC, _NS = 2, 16
_NW = _NC * _NS
_BPW = B // _NW
_IPW = _BPW * F
_IPWG = _BPW * FP
_CH = 128
_NCHW = _IPW // _CH
_NCHG = _IPWG // _CH
_GB = _BPW * 8


def _sc_gather_body(idxg_hbm, idxw_hbm, t0, t1, t2, t3, w1,
                    u_out, wsel_out,
                    idxg_v, idxw_v, rows_v, wsel_v, sem, wsem):
    wid = lax.axis_index("s") * _NC + lax.axis_index("c")
    pltpu.sync_copy(idxg_hbm.at[wid], idxg_v)
    pltpu.sync_copy(idxw_hbm.at[wid], idxw_v)

    def wbody(j, carry):
        pltpu.async_copy(w1.at[idxw_v.at[j]],
                         wsel_v.at[pl.ds(j * _CH, _CH)], wsem)
        return carry
    lax.fori_loop(0, _NCHW, wbody, 0)

    for slot, tref in enumerate((t0, t1, t2, t3)):
        def tbody(j, carry, tref=tref):
            pltpu.async_copy(tref.at[idxg_v.at[j]],
                             rows_v.at[pl.ds(j * _CH, _CH)], sem)
            return carry
        lax.fori_loop(0, _NCHG, tbody, 0)

        def twait(j, carry, tref=tref):
            pltpu.make_async_copy(tref.at[idxg_v.at[j]],
                                  rows_v.at[pl.ds(j * _CH, _CH)], sem).wait()
            return carry
        lax.fori_loop(0, _NCHG, twait, 0)
        for g in range(4):
            pltpu.sync_copy(rows_v.at[pl.ds(g * _GB, _GB)],
                            u_out.at[slot, g, pl.ds(wid * _GB, _GB)])

    def wwait(j, carry):
        pltpu.make_async_copy(w1.at[idxw_v.at[j]],
                              wsel_v.at[pl.ds(j * _CH, _CH)], wsem).wait()
        return carry
    lax.fori_loop(0, _NCHW, wwait, 0)
    pltpu.sync_copy(wsel_v, wsel_out.at[pl.ds(wid * _IPW, _IPW)])


@functools.cache
def _sc_gather():
    return pl.kernel(
        _sc_gather_body,
        mesh=plsc.VectorSubcoreMesh(core_axis_name="c", subcore_axis_name="s",
                                    num_cores=_NC, num_subcores=_NS),
        compiler_params=pltpu.CompilerParams(use_tc_tiling_on_sc=False),
        out_type=[
            jax.ShapeDtypeStruct((4, 4, B * 8, E), jnp.float32),
            jax.ShapeDtypeStruct((B * F,), jnp.float32),
        ],
        scratch_types=[
            pltpu.VMEM((_NCHG, _CH), jnp.int32),
            pltpu.VMEM((_NCHW, _CH), jnp.int32),
            pltpu.VMEM((_IPWG, E), jnp.float32),
            pltpu.VMEM((_IPW,), jnp.float32),
            pltpu.SemaphoreType.DMA,
            pltpu.SemaphoreType.DMA,
        ],
    )


_BT = 1024


def _tc_body(xw_ref, u_ref, q_ref, b_ref, o_ref):
    acc = jnp.sum(xw_ref[...], axis=1, keepdims=True) + b_ref[0, 0]
    for t in range(4):
        y = None
        for g in range(4):
            yg = lax.dot_general(u_ref[t, g], q_ref[t, g],
                                 (((1,), (0,)), ((), ())),
                                 precision=lax.Precision.HIGHEST,
                                 preferred_element_type=jnp.float32)
            y = yg if y is None else y + yg
        for g in range(4):
            acc = acc + jnp.sum(y[:, g * 128:(g + 1) * 128] * u_ref[t, g],
                                axis=1, keepdims=True)
    o_ref[...] = acc


def kernel(inputs, w, b, v0, v1, v2, v3, kernel_vec, kernel_num, kernel_mat):
    idx32 = inputs.astype(jnp.int32)
    idxw = idx32.reshape(_NW, _NCHW, _CH)
    idxp = jnp.concatenate(
        [idx32, idx32[:, :FP - F]], axis=1)
    idxg = (idxp.reshape(_NW, _BPW, 4, 8)
            .transpose(0, 2, 1, 3)
            .reshape(_NW, _NCHG, _CH))

    i16 = jnp.asarray(_EYE_E)
    tabs = [_repack(v.T, i16) for v in (v0, v1, v2, v3)]
    u_flat, xw_flat = _sc_gather()(
        idxg, idxw, *[t.reshape(V, E) for t in tabs], w)
    u4 = u_flat.reshape(4, 4, B, 128)
    xw = xw_flat.reshape(B, F)

    a1 = (kernel_vec[0].T @ _PSCAT).reshape(E, F, F)
    q1 = jnp.einsum('eij,ef->iejf', a1, _EYE_E).reshape(FE, FE)
    a2 = (kernel_num[0, :, 0] @ _PSCAT).reshape(F, F)
    q2 = jnp.einsum('ij,ef->iejf', a2, _EYE_E).reshape(FE, FE)
    a3 = (_PSCAT.T @ kernel_mat.reshape(_NP, E * E)).reshape(F, F, E, E)
    q3 = a3.transpose(0, 2, 1, 3).reshape(FE, FE)
    qs = jnp.stack([jnp.asarray(_Q0), q1, q2, q3])
    qp = jnp.pad(qs, ((0, 0), (0, FEP - FE), (0, FEP - FE)))
    qg = qp.reshape(4, 4, 128, FEP)

    logits = pl.pallas_call(
        _tc_body,
        grid=(B // _BT,),
        out_shape=jax.ShapeDtypeStruct((B, 1), jnp.float32),
        in_specs=[
            pl.BlockSpec((_BT, F), lambda i: (i, 0)),
            pl.BlockSpec((4, 4, _BT, 128), lambda i: (0, 0, i, 0)),
            pl.BlockSpec((4, 4, 128, FEP), lambda i: (0, 0, 0, 0)),
            pl.BlockSpec(memory_space=pltpu.SMEM),
        ],
        out_specs=pl.BlockSpec((_BT, 1), lambda i: (i, 0)),
    )(xw, u4, qg, b.reshape(1, 1))
    return logits


# speedup vs baseline: 1.0590x; 1.0590x over previous
import functools

import numpy as np
from itertools import combinations

import jax
import jax.numpy as jnp
from jax import lax
from jax.experimental import pallas as pl
from jax.experimental.pallas import tpu as pltpu
from jax.experimental.pallas import tpu_sc as plsc

F = 26
E = 16
FP = 32
FE = F * E
FEP = FP * E
B = 4096
V = 1000000

_PAIRS = list(combinations(range(F), 2))
_LEFT = np.array([p[0] for p in _PAIRS], dtype=np.int32)
_RIGHT = np.array([p[1] for p in _PAIRS], dtype=np.int32)
_NP = len(_PAIRS)

_PSCAT = np.zeros((_NP, F * F), np.float32)
_PSCAT[np.arange(_NP), _LEFT * F + _RIGHT] = 1.0
_EYE_E = np.eye(E, dtype=np.float32)
_Q0 = np.einsum('ij,ef->iejf', np.triu(np.ones((F, F), np.float32), 1),
                _EYE_E).reshape(FE, FE)


_CB = 16384
_CBR = _CB // 8


def _repack_body(x_ref, i_ref, o_ref):
    z = lax.dot_general(x_ref[...], i_ref[...], (((0,), (0,)), ((), ())),
                        preferred_element_type=jnp.float32)
    z3 = z.reshape(_CBR, 8, 16)
    o_ref[...] = jnp.concatenate([z3[:, fo, :] for fo in range(8)], axis=1)


def _repack(vt, i16):
    nb = (V + _CB - 1) // _CB
    return pl.pallas_call(
        _repack_body,
        grid=(nb,),
        out_shape=jax.ShapeDtypeStruct((V // 8, 128), jnp.float32),
        in_specs=[pl.BlockSpec((E, _CB), lambda i: (0, i)),
                  pl.BlockSpec((E, E), lambda i: (0, 0))],
        out_specs=pl.BlockSpec((_CBR, 128), lambda i: (i, 0)),
    )(vt, i16)


_NC, _NS = 2, 16
_NW = _NC * _NS
_BPW = B // _NW
_IPW = _BPW * F
_IPWG = _BPW * FP
_CH = 128
_NCHW = _IPW // _CH
_NCHG = _IPWG // _CH
_GB = _BPW * 8


def _sc_gather_body(idxg_hbm, idxw_hbm, t0, t1, t2, t3, w1,
                    u_out, wsel_out,
                    idxg_v, idxw_v, rows_v, wsel_v, sem, wsem):
    wid = lax.axis_index("s") * _NC + lax.axis_index("c")
    pltpu.sync_copy(idxg_hbm.at[wid], idxg_v)
    pltpu.sync_copy(idxw_hbm.at[wid], idxw_v)

    def wbody(j, carry):
        pltpu.async_copy(w1.at[idxw_v.at[j]],
                         wsel_v.at[pl.ds(j * _CH, _CH)], wsem)
        return carry
    lax.fori_loop(0, _NCHW, wbody, 0)

    for slot, tref in enumerate((t0, t1, t2, t3)):
        def tbody(j, carry, tref=tref):
            pltpu.async_copy(tref.at[idxg_v.at[j]],
                             rows_v.at[pl.ds(j * _CH, _CH)], sem)
            return carry
        lax.fori_loop(0, _NCHG, tbody, 0)

        def twait(j, carry, tref=tref):
            pltpu.make_async_copy(tref.at[idxg_v.at[j]],
                                  rows_v.at[pl.ds(j * _CH, _CH)], sem).wait()
            return carry
        lax.fori_loop(0, _NCHG, twait, 0)
        for g in range(4):
            pltpu.sync_copy(rows_v.at[pl.ds(g * _GB, _GB)],
                            u_out.at[slot, g, pl.ds(wid * _GB, _GB)])

    def wwait(j, carry):
        pltpu.make_async_copy(w1.at[idxw_v.at[j]],
                              wsel_v.at[pl.ds(j * _CH, _CH)], wsem).wait()
        return carry
    lax.fori_loop(0, _NCHW, wwait, 0)
    pltpu.sync_copy(wsel_v, wsel_out.at[pl.ds(wid * _IPW, _IPW)])


@functools.cache
def _sc_gather():
    return pl.kernel(
        _sc_gather_body,
        mesh=plsc.VectorSubcoreMesh(core_axis_name="c", subcore_axis_name="s",
                                    num_cores=_NC, num_subcores=_NS),
        compiler_params=pltpu.CompilerParams(use_tc_tiling_on_sc=False),
        out_type=[
            jax.ShapeDtypeStruct((4, 4, B * 8, E), jnp.float32),
            jax.ShapeDtypeStruct((B * F,), jnp.float32),
        ],
        scratch_types=[
            pltpu.VMEM((_NCHG, _CH), jnp.int32),
            pltpu.VMEM((_NCHW, _CH), jnp.int32),
            pltpu.VMEM((_IPWG, E), jnp.float32),
            pltpu.VMEM((_IPW,), jnp.float32),
            pltpu.SemaphoreType.DMA,
            pltpu.SemaphoreType.DMA,
        ],
    )


_BT = 1024


def _tc_body(xw_ref, u_ref, q_ref, b_ref, o_ref):
    acc = jnp.sum(xw_ref[...], axis=1, keepdims=True) + b_ref[0, 0]
    for t in range(4):
        y = None
        for g in range(4):
            yg = lax.dot_general(u_ref[t, g], q_ref[t, g],
                                 (((1,), (0,)), ((), ())),
                                 precision=lax.Precision.HIGHEST,
                                 preferred_element_type=jnp.float32)
            y = yg if y is None else y + yg
        for g in range(4):
            acc = acc + jnp.sum(y[:, g * 128:(g + 1) * 128] * u_ref[t, g],
                                axis=1, keepdims=True)
    o_ref[...] = acc


def kernel(inputs, w, b, v0, v1, v2, v3, kernel_vec, kernel_num, kernel_mat):
    idx32 = inputs.astype(jnp.int32)
    idxw = idx32.reshape(_NW, _NCHW, _CH)
    idxp = jnp.concatenate(
        [idx32, idx32[:, :FP - F]], axis=1)
    idxg = (idxp.reshape(_NW, _BPW, 4, 8)
            .transpose(0, 2, 1, 3)
            .reshape(_NW, _NCHG, _CH))

    i16 = jnp.asarray(_EYE_E)
    tabs = [_repack(v.T, i16) for v in (v0, v1, v2, v3)]
    u_flat, xw_flat = _sc_gather()(
        idxg, idxw, *[t.reshape(V, E) for t in tabs], w)
    u4 = u_flat.reshape(4, 4, B, 128)
    xw = xw_flat.reshape(B, F)

    a1 = (kernel_vec[0].T @ _PSCAT).reshape(E, F, F)
    q1 = jnp.einsum('eij,ef->iejf', a1, _EYE_E).reshape(FE, FE)
    a2 = (kernel_num[0, :, 0] @ _PSCAT).reshape(F, F)
    q2 = jnp.einsum('ij,ef->iejf', a2, _EYE_E).reshape(FE, FE)
    a3 = (_PSCAT.T @ kernel_mat.reshape(_NP, E * E)).reshape(F, F, E, E)
    q3 = a3.transpose(0, 2, 1, 3).reshape(FE, FE)
    qs = jnp.stack([jnp.asarray(_Q0), q1, q2, q3])
    qp = jnp.pad(qs, ((0, 0), (0, FEP - FE), (0, FEP - FE)))
    qg = qp.reshape(4, 4, 128, FEP)

    logits = pl.pallas_call(
        _tc_body,
        grid=(B // _BT,),
        out_shape=jax.ShapeDtypeStruct((B, 1), jnp.float32),
        in_specs=[
            pl.BlockSpec((_BT, F), lambda i: (i, 0)),
            pl.BlockSpec((4, 4, _BT, 128), lambda i: (0, 0, i, 0)),
            pl.BlockSpec((4, 4, 128, FEP), lambda i: (0, 0, 0, 0)),
            pl.BlockSpec(memory_space=pltpu.SMEM),
        ],
        out_specs=pl.BlockSpec((_BT, 1), lambda i: (i, 0)),
    )(xw, u4, qg, b.reshape(1, 1))
    return logits


# speedup vs baseline: 2.9511x; 2.7867x over previous
import functools

import numpy as np
from itertools import combinations

import jax
import jax.numpy as jnp
from jax import lax
from jax.experimental import pallas as pl
from jax.experimental.pallas import tpu as pltpu
from jax.experimental.pallas import tpu_sc as plsc

F = 26
E = 16
FP = 32
FE = F * E
FEP = FP * E
B = 4096
V = 1000000

_PAIRS = list(combinations(range(F), 2))
_LEFT = np.array([p[0] for p in _PAIRS], dtype=np.int32)
_RIGHT = np.array([p[1] for p in _PAIRS], dtype=np.int32)
_NP = len(_PAIRS)

_PSCAT = np.zeros((_NP, F * F), np.float32)
_PSCAT[np.arange(_NP), _LEFT * F + _RIGHT] = 1.0
_EYE_E = np.eye(E, dtype=np.float32)
_Q0 = np.einsum('ij,ef->iejf', np.triu(np.ones((F, F), np.float32), 1),
                _EYE_E).reshape(FE, FE)


_CB = 16384
_CBR = _CB // 8
_NB = (V + _CB - 1) // _CB
_VP = _NB * _CB


def _repack_body(x_ref, i_ref, o_ref):
    x = x_ref[...]
    col = lax.broadcasted_iota(jnp.int32, (E, _CB), 1) + pl.program_id(0) * _CB
    x = jnp.where(col < V, x, 0.0)
    xp = jnp.concatenate(
        [x[:, q * _CBR:(q + 1) * _CBR] for q in range(8)], axis=0
    )
    o_ref[...] = lax.dot_general(xp, i_ref[...], (((0,), (0,)), ((), ())),
                                 preferred_element_type=jnp.float32)


def _repack(vt, i128):
    return pl.pallas_call(
        _repack_body,
        grid=(_NB,),
        out_shape=jax.ShapeDtypeStruct((_NB * _CBR, 128), jnp.float32),
        in_specs=[pl.BlockSpec((E, _CB), lambda i: (0, i)),
                  pl.BlockSpec((128, 128), lambda i: (0, 0))],
        out_specs=pl.BlockSpec((_CBR, 128), lambda i: (i, 0)),
    )(vt, i128)


def _remap_idx(r):
    loc = r & (_CB - 1)
    return (r - loc) + ((loc & (_CBR - 1)) << 3) + (loc >> 11)


_NC, _NS = 2, 16
_NW = _NC * _NS
_BPW = B // _NW
_IPW = _BPW * F
_IPWG = _BPW * FP
_CH = 128
_NCHW = _IPW // _CH
_NCHG = _IPWG // _CH
_GB = _BPW * 8


def _sc_gather_body(idxg_hbm, idxw_hbm, t0, t1, t2, t3, w1,
                    u_out, wsel_out,
                    idxg_v, idxw_v, rows_v, wsel_v, sem, wsem):
    wid = lax.axis_index("s") * _NC + lax.axis_index("c")
    pltpu.sync_copy(idxg_hbm.at[wid], idxg_v)
    pltpu.sync_copy(idxw_hbm.at[wid], idxw_v)

    def wbody(j, carry):
        pltpu.async_copy(w1.at[idxw_v.at[j]],
                         wsel_v.at[pl.ds(j * _CH, _CH)], wsem)
        return carry
    lax.fori_loop(0, _NCHW, wbody, 0)

    for slot, tref in enumerate((t0, t1, t2, t3)):
        def tbody(j, carry, tref=tref):
            pltpu.async_copy(tref.at[idxg_v.at[j]],
                             rows_v.at[pl.ds(j * _CH, _CH)], sem)
            return carry
        lax.fori_loop(0, _NCHG, tbody, 0)

        def twait(j, carry, tref=tref):
            pltpu.make_async_copy(tref.at[idxg_v.at[j]],
                                  rows_v.at[pl.ds(j * _CH, _CH)], sem).wait()
            return carry
        lax.fori_loop(0, _NCHG, twait, 0)
        for g in range(4):
            pltpu.sync_copy(rows_v.at[pl.ds(g * _GB, _GB)],
                            u_out.at[slot, g, pl.ds(wid * _GB, _GB)])

    def wwait(j, carry):
        pltpu.make_async_copy(w1.at[idxw_v.at[j]],
                              wsel_v.at[pl.ds(j * _CH, _CH)], wsem).wait()
        return carry
    lax.fori_loop(0, _NCHW, wwait, 0)
    pltpu.sync_copy(wsel_v, wsel_out.at[pl.ds(wid * _IPW, _IPW)])


@functools.cache
def _sc_gather():
    return pl.kernel(
        _sc_gather_body,
        mesh=plsc.VectorSubcoreMesh(core_axis_name="c", subcore_axis_name="s",
                                    num_cores=_NC, num_subcores=_NS),
        compiler_params=pltpu.CompilerParams(use_tc_tiling_on_sc=False),
        out_type=[
            jax.ShapeDtypeStruct((4, 4, B * 8, E), jnp.float32),
            jax.ShapeDtypeStruct((B * F,), jnp.float32),
        ],
        scratch_types=[
            pltpu.VMEM((_NCHG, _CH), jnp.int32),
            pltpu.VMEM((_NCHW, _CH), jnp.int32),
            pltpu.VMEM((_IPWG, E), jnp.float32),
            pltpu.VMEM((_IPW,), jnp.float32),
            pltpu.SemaphoreType.DMA,
            pltpu.SemaphoreType.DMA,
        ],
    )


_BT = 1024


def _tc_body(xw_ref, u_ref, q_ref, b_ref, o_ref):
    acc = jnp.sum(xw_ref[...], axis=1, keepdims=True) + b_ref[0, 0]
    for t in range(4):
        y = None
        for g in range(4):
            yg = lax.dot_general(u_ref[t, g], q_ref[t, g],
                                 (((1,), (0,)), ((), ())),
                                 precision=lax.Precision.HIGHEST,
                                 preferred_element_type=jnp.float32)
            y = yg if y is None else y + yg
        for g in range(4):
            acc = acc + jnp.sum(y[:, g * 128:(g + 1) * 128] * u_ref[t, g],
                                axis=1, keepdims=True)
    o_ref[...] = acc


def kernel(inputs, w, b, v0, v1, v2, v3, kernel_vec, kernel_num, kernel_mat):
    idx32 = inputs.astype(jnp.int32)
    idxw = idx32.reshape(_NW, _NCHW, _CH)
    idxp = _remap_idx(jnp.concatenate(
        [idx32, idx32[:, :FP - F]], axis=1))
    idxg = (idxp.reshape(_NW, _BPW, 4, 8)
            .transpose(0, 2, 1, 3)
            .reshape(_NW, _NCHG, _CH))

    i128 = jnp.asarray(np.eye(128, dtype=np.float32))
    tabs = [_repack(v.T, i128) for v in (v0, v1, v2, v3)]
    u_flat, xw_flat = _sc_gather()(
        idxg, idxw, *[t.reshape(_VP, E) for t in tabs], w)
    u4 = u_flat.reshape(4, 4, B, 128)
    xw = xw_flat.reshape(B, F)

    a1 = (kernel_vec[0].T @ _PSCAT).reshape(E, F, F)
    q1 = jnp.einsum('eij,ef->iejf', a1, _EYE_E).reshape(FE, FE)
    a2 = (kernel_num[0, :, 0] @ _PSCAT).reshape(F, F)
    q2 = jnp.einsum('ij,ef->iejf', a2, _EYE_E).reshape(FE, FE)
    a3 = (_PSCAT.T @ kernel_mat.reshape(_NP, E * E)).reshape(F, F, E, E)
    q3 = a3.transpose(0, 2, 1, 3).reshape(FE, FE)
    qs = jnp.stack([jnp.asarray(_Q0), q1, q2, q3])
    qp = jnp.pad(qs, ((0, 0), (0, FEP - FE), (0, FEP - FE)))
    qg = qp.reshape(4, 4, 128, FEP)

    logits = pl.pallas_call(
        _tc_body,
        grid=(B // _BT,),
        out_shape=jax.ShapeDtypeStruct((B, 1), jnp.float32),
        in_specs=[
            pl.BlockSpec((_BT, F), lambda i: (i, 0)),
            pl.BlockSpec((4, 4, _BT, 128), lambda i: (0, 0, i, 0)),
            pl.BlockSpec((4, 4, 128, FEP), lambda i: (0, 0, 0, 0)),
            pl.BlockSpec(memory_space=pltpu.SMEM),
        ],
        out_specs=pl.BlockSpec((_BT, 1), lambda i: (i, 0)),
    )(xw, u4, qg, b.reshape(1, 1))
    return logits


# speedup vs baseline: 3.8365x; 1.3000x over previous
import functools

import numpy as np
from itertools import combinations

import jax
import jax.numpy as jnp
from jax import lax
from jax.experimental import pallas as pl
from jax.experimental.pallas import tpu as pltpu
from jax.experimental.pallas import tpu_sc as plsc

F = 26
E = 16
FP = 32
FE = F * E
FEP = FP * E
B = 4096
V = 1000000

_PAIRS = list(combinations(range(F), 2))
_LEFT = np.array([p[0] for p in _PAIRS], dtype=np.int32)
_RIGHT = np.array([p[1] for p in _PAIRS], dtype=np.int32)
_NP = len(_PAIRS)

_PSCAT = np.zeros((_NP, F * F), np.float32)
_PSCAT[np.arange(_NP), _LEFT * F + _RIGHT] = 1.0
_EYE_E = np.eye(E, dtype=np.float32)
_Q0 = np.einsum('ij,ef->iejf', np.triu(np.ones((F, F), np.float32), 1),
                _EYE_E).reshape(FE, FE)


_CB = 65536
_CBR = _CB // 8
_NB = (V + _CB - 1) // _CB
_VP = _NB * _CB


def _repack_body(x_ref, i_ref, o_ref):
    x = x_ref[...]
    col = lax.broadcasted_iota(jnp.int32, (E, _CB), 1) + pl.program_id(0) * _CB
    x = jnp.where(col < V, x, 0.0)
    xp = jnp.concatenate(
        [x[:, q * _CBR:(q + 1) * _CBR] for q in range(8)], axis=0
    )
    o_ref[...] = lax.dot_general(xp, i_ref[...], (((0,), (0,)), ((), ())),
                                 preferred_element_type=jnp.float32)


def _repack(vt, i128):
    return pl.pallas_call(
        _repack_body,
        grid=(_NB,),
        out_shape=jax.ShapeDtypeStruct((_NB * _CBR, 128), jnp.float32),
        in_specs=[pl.BlockSpec((E, _CB), lambda i: (0, i)),
                  pl.BlockSpec((128, 128), lambda i: (0, 0))],
        out_specs=pl.BlockSpec((_CBR, 128), lambda i: (i, 0)),
    )(vt, i128)


_CBR_BITS = int(np.log2(_CBR))


def _remap_idx(r):
    loc = r & (_CB - 1)
    return (r - loc) + ((loc & (_CBR - 1)) << 3) + (loc >> _CBR_BITS)


_NC, _NS = 2, 16
_NW = _NC * _NS
_BPW = B // _NW
_IPW = _BPW * F
_IPWG = _BPW * FP
_CH = 128
_NCHW = _IPW // _CH
_NCHG = _IPWG // _CH
_GB = _BPW * 8


def _sc_gather_body(idxg_hbm, idxw_hbm, t0, t1, t2, t3, w1,
                    u_out, wsel_out,
                    idxg_v, idxw_v, rows_v, wsel_v, sem, wsem):
    wid = lax.axis_index("s") * _NC + lax.axis_index("c")
    pltpu.sync_copy(idxg_hbm.at[wid], idxg_v)
    pltpu.sync_copy(idxw_hbm.at[wid], idxw_v)

    def wbody(j, carry):
        pltpu.async_copy(w1.at[idxw_v.at[j]],
                         wsel_v.at[pl.ds(j * _CH, _CH)], wsem)
        return carry
    lax.fori_loop(0, _NCHW, wbody, 0)

    for slot, tref in enumerate((t0, t1, t2, t3)):
        def tbody(j, carry, tref=tref):
            pltpu.async_copy(tref.at[idxg_v.at[j]],
                             rows_v.at[pl.ds(j * _CH, _CH)], sem)
            return carry
        lax.fori_loop(0, _NCHG, tbody, 0)

        def twait(j, carry, tref=tref):
            pltpu.make_async_copy(tref.at[idxg_v.at[j]],
                                  rows_v.at[pl.ds(j * _CH, _CH)], sem).wait()
            return carry
        lax.fori_loop(0, _NCHG, twait, 0)
        for g in range(4):
            pltpu.sync_copy(rows_v.at[pl.ds(g * _GB, _GB)],
                            u_out.at[slot, g, pl.ds(wid * _GB, _GB)])

    def wwait(j, carry):
        pltpu.make_async_copy(w1.at[idxw_v.at[j]],
                              wsel_v.at[pl.ds(j * _CH, _CH)], wsem).wait()
        return carry
    lax.fori_loop(0, _NCHW, wwait, 0)
    pltpu.sync_copy(wsel_v, wsel_out.at[pl.ds(wid * _IPW, _IPW)])


@functools.cache
def _sc_gather():
    return pl.kernel(
        _sc_gather_body,
        mesh=plsc.VectorSubcoreMesh(core_axis_name="c", subcore_axis_name="s",
                                    num_cores=_NC, num_subcores=_NS),
        compiler_params=pltpu.CompilerParams(use_tc_tiling_on_sc=False),
        out_type=[
            jax.ShapeDtypeStruct((4, 4, B * 8, E), jnp.float32),
            jax.ShapeDtypeStruct((B * F,), jnp.float32),
        ],
        scratch_types=[
            pltpu.VMEM((_NCHG, _CH), jnp.int32),
            pltpu.VMEM((_NCHW, _CH), jnp.int32),
            pltpu.VMEM((_IPWG, E), jnp.float32),
            pltpu.VMEM((_IPW,), jnp.float32),
            pltpu.SemaphoreType.DMA,
            pltpu.SemaphoreType.DMA,
        ],
    )


_BT = 1024


def _tc_body(xw_ref, u_ref, q_ref, b_ref, o_ref):
    acc = jnp.sum(xw_ref[...], axis=1, keepdims=True) + b_ref[0, 0]
    for t in range(4):
        y = None
        for g in range(4):
            yg = lax.dot_general(u_ref[t, g], q_ref[t, g],
                                 (((1,), (0,)), ((), ())),
                                 precision=lax.Precision.HIGHEST,
                                 preferred_element_type=jnp.float32)
            y = yg if y is None else y + yg
        for g in range(4):
            acc = acc + jnp.sum(y[:, g * 128:(g + 1) * 128] * u_ref[t, g],
                                axis=1, keepdims=True)
    o_ref[...] = acc


def kernel(inputs, w, b, v0, v1, v2, v3, kernel_vec, kernel_num, kernel_mat):
    idx32 = inputs.astype(jnp.int32)
    idxw = idx32.reshape(_NW, _NCHW, _CH)
    idxp = _remap_idx(jnp.concatenate(
        [idx32, idx32[:, :FP - F]], axis=1))
    idxg = (idxp.reshape(_NW, _BPW, 4, 8)
            .transpose(0, 2, 1, 3)
            .reshape(_NW, _NCHG, _CH))

    i128 = jnp.asarray(np.eye(128, dtype=np.float32))
    tabs = [_repack(v.T, i128) for v in (v0, v1, v2, v3)]
    u_flat, xw_flat = _sc_gather()(
        idxg, idxw, *[t.reshape(_VP, E) for t in tabs], w)
    u4 = u_flat.reshape(4, 4, B, 128)
    xw = xw_flat.reshape(B, F)

    a1 = (kernel_vec[0].T @ _PSCAT).reshape(E, F, F)
    q1 = jnp.einsum('eij,ef->iejf', a1, _EYE_E).reshape(FE, FE)
    a2 = (kernel_num[0, :, 0] @ _PSCAT).reshape(F, F)
    q2 = jnp.einsum('ij,ef->iejf', a2, _EYE_E).reshape(FE, FE)
    a3 = (_PSCAT.T @ kernel_mat.reshape(_NP, E * E)).reshape(F, F, E, E)
    q3 = a3.transpose(0, 2, 1, 3).reshape(FE, FE)
    qs = jnp.stack([jnp.asarray(_Q0), q1, q2, q3])
    qp = jnp.pad(qs, ((0, 0), (0, FEP - FE), (0, FEP - FE)))
    qg = qp.reshape(4, 4, 128, FEP)

    logits = pl.pallas_call(
        _tc_body,
        grid=(B // _BT,),
        out_shape=jax.ShapeDtypeStruct((B, 1), jnp.float32),
        in_specs=[
            pl.BlockSpec((_BT, F), lambda i: (i, 0)),
            pl.BlockSpec((4, 4, _BT, 128), lambda i: (0, 0, i, 0)),
            pl.BlockSpec((4, 4, 128, FEP), lambda i: (0, 0, 0, 0)),
            pl.BlockSpec(memory_space=pltpu.SMEM),
        ],
        out_specs=pl.BlockSpec((_BT, 1), lambda i: (i, 0)),
    )(xw, u4, qg, b.reshape(1, 1))
    return logits


# speedup vs baseline: 3.8778x; 1.0108x over previous
import functools

import numpy as np
from itertools import combinations

import jax
import jax.numpy as jnp
from jax import lax
from jax.experimental import pallas as pl
from jax.experimental.pallas import tpu as pltpu
from jax.experimental.pallas import tpu_sc as plsc

F = 26
E = 16
FP = 32
FE = F * E
FEP = FP * E
B = 4096
V = 1000000

_PAIRS = list(combinations(range(F), 2))
_LEFT = np.array([p[0] for p in _PAIRS], dtype=np.int32)
_RIGHT = np.array([p[1] for p in _PAIRS], dtype=np.int32)
_NP = len(_PAIRS)

_PSCAT = np.zeros((_NP, F * F), np.float32)
_PSCAT[np.arange(_NP), _LEFT * F + _RIGHT] = 1.0
_EYE_E = np.eye(E, dtype=np.float32)
_Q0 = np.einsum('ij,ef->iejf', np.triu(np.ones((F, F), np.float32), 1),
                _EYE_E).reshape(FE, FE)


_CB = 131072
_CBR = _CB // 8
_NB = (V + _CB - 1) // _CB
_VP = _NB * _CB


def _repack_body(x_ref, i_ref, o_ref):
    x = x_ref[...]
    col = lax.broadcasted_iota(jnp.int32, (E, _CB), 1) + pl.program_id(0) * _CB
    x = jnp.where(col < V, x, 0.0)
    xp = jnp.concatenate(
        [x[:, q * _CBR:(q + 1) * _CBR] for q in range(8)], axis=0
    )
    o_ref[...] = lax.dot_general(xp, i_ref[...], (((0,), (0,)), ((), ())),
                                 preferred_element_type=jnp.float32)


def _repack(vt, i128):
    return pl.pallas_call(
        _repack_body,
        grid=(_NB,),
        out_shape=jax.ShapeDtypeStruct((_NB * _CBR, 128), jnp.float32),
        in_specs=[pl.BlockSpec((E, _CB), lambda i: (0, i)),
                  pl.BlockSpec((128, 128), lambda i: (0, 0))],
        out_specs=pl.BlockSpec((_CBR, 128), lambda i: (i, 0)),
    )(vt, i128)


_CBR_BITS = int(np.log2(_CBR))


def _remap_idx(r):
    loc = r & (_CB - 1)
    return (r - loc) + ((loc & (_CBR - 1)) << 3) + (loc >> _CBR_BITS)


_NC, _NS = 2, 16
_NW = _NC * _NS
_BPW = B // _NW
_IPW = _BPW * F
_IPWG = _BPW * FP
_CH = 128
_NCHW = _IPW // _CH
_NCHG = _IPWG // _CH
_GB = _BPW * 8


def _sc_gather_body(idxg_hbm, idxw_hbm, t0, t1, t2, t3, w1,
                    u_out, wsel_out,
                    idxg_v, idxw_v, rows_v, wsel_v, sem, wsem):
    wid = lax.axis_index("s") * _NC + lax.axis_index("c")
    pltpu.sync_copy(idxg_hbm.at[wid], idxg_v)
    pltpu.sync_copy(idxw_hbm.at[wid], idxw_v)

    def wbody(j, carry):
        pltpu.async_copy(w1.at[idxw_v.at[j]],
                         wsel_v.at[pl.ds(j * _CH, _CH)], wsem)
        return carry
    lax.fori_loop(0, _NCHW, wbody, 0)

    for slot, tref in enumerate((t0, t1, t2, t3)):
        def tbody(j, carry, tref=tref):
            pltpu.async_copy(tref.at[idxg_v.at[j]],
                             rows_v.at[pl.ds(j * _CH, _CH)], sem)
            return carry
        lax.fori_loop(0, _NCHG, tbody, 0)

        def twait(j, carry, tref=tref):
            pltpu.make_async_copy(tref.at[idxg_v.at[j]],
                                  rows_v.at[pl.ds(j * _CH, _CH)], sem).wait()
            return carry
        lax.fori_loop(0, _NCHG, twait, 0)
        for g in range(4):
            pltpu.sync_copy(rows_v.at[pl.ds(g * _GB, _GB)],
                            u_out.at[slot, g, pl.ds(wid * _GB, _GB)])

    def wwait(j, carry):
        pltpu.make_async_copy(w1.at[idxw_v.at[j]],
                              wsel_v.at[pl.ds(j * _CH, _CH)], wsem).wait()
        return carry
    lax.fori_loop(0, _NCHW, wwait, 0)
    pltpu.sync_copy(wsel_v, wsel_out.at[pl.ds(wid * _IPW, _IPW)])


@functools.cache
def _sc_gather():
    return pl.kernel(
        _sc_gather_body,
        mesh=plsc.VectorSubcoreMesh(core_axis_name="c", subcore_axis_name="s",
                                    num_cores=_NC, num_subcores=_NS),
        compiler_params=pltpu.CompilerParams(use_tc_tiling_on_sc=False),
        out_type=[
            jax.ShapeDtypeStruct((4, 4, B * 8, E), jnp.float32),
            jax.ShapeDtypeStruct((B * F,), jnp.float32),
        ],
        scratch_types=[
            pltpu.VMEM((_NCHG, _CH), jnp.int32),
            pltpu.VMEM((_NCHW, _CH), jnp.int32),
            pltpu.VMEM((_IPWG, E), jnp.float32),
            pltpu.VMEM((_IPW,), jnp.float32),
            pltpu.SemaphoreType.DMA,
            pltpu.SemaphoreType.DMA,
        ],
    )


_BT = 1024


def _tc_body(xw_ref, u_ref, q_ref, b_ref, o_ref):
    acc = jnp.sum(xw_ref[...], axis=1, keepdims=True) + b_ref[0, 0]
    for t in range(4):
        y = None
        for g in range(4):
            yg = lax.dot_general(u_ref[t, g], q_ref[t, g],
                                 (((1,), (0,)), ((), ())),
                                 precision=lax.Precision.HIGHEST,
                                 preferred_element_type=jnp.float32)
            y = yg if y is None else y + yg
        for g in range(4):
            acc = acc + jnp.sum(y[:, g * 128:(g + 1) * 128] * u_ref[t, g],
                                axis=1, keepdims=True)
    o_ref[...] = acc


def kernel(inputs, w, b, v0, v1, v2, v3, kernel_vec, kernel_num, kernel_mat):
    idx32 = inputs.astype(jnp.int32)
    idxw = idx32.reshape(_NW, _NCHW, _CH)
    idxp = _remap_idx(jnp.concatenate(
        [idx32, idx32[:, :FP - F]], axis=1))
    idxg = (idxp.reshape(_NW, _BPW, 4, 8)
            .transpose(0, 2, 1, 3)
            .reshape(_NW, _NCHG, _CH))

    i128 = jnp.asarray(np.eye(128, dtype=np.float32))
    tabs = [_repack(v.T, i128) for v in (v0, v1, v2, v3)]
    u_flat, xw_flat = _sc_gather()(
        idxg, idxw, *[t.reshape(_VP, E) for t in tabs], w)
    u4 = u_flat.reshape(4, 4, B, 128)
    xw = xw_flat.reshape(B, F)

    a1 = (kernel_vec[0].T @ _PSCAT).reshape(E, F, F)
    q1 = jnp.einsum('eij,ef->iejf', a1, _EYE_E).reshape(FE, FE)
    a2 = (kernel_num[0, :, 0] @ _PSCAT).reshape(F, F)
    q2 = jnp.einsum('ij,ef->iejf', a2, _EYE_E).reshape(FE, FE)
    a3 = (_PSCAT.T @ kernel_mat.reshape(_NP, E * E)).reshape(F, F, E, E)
    q3 = a3.transpose(0, 2, 1, 3).reshape(FE, FE)
    qs = jnp.stack([jnp.asarray(_Q0), q1, q2, q3])
    qp = jnp.pad(qs, ((0, 0), (0, FEP - FE), (0, FEP - FE)))
    qg = qp.reshape(4, 4, 128, FEP)

    logits = pl.pallas_call(
        _tc_body,
        grid=(B // _BT,),
        out_shape=jax.ShapeDtypeStruct((B, 1), jnp.float32),
        in_specs=[
            pl.BlockSpec((_BT, F), lambda i: (i, 0)),
            pl.BlockSpec((4, 4, _BT, 128), lambda i: (0, 0, i, 0)),
            pl.BlockSpec((4, 4, 128, FEP), lambda i: (0, 0, 0, 0)),
            pl.BlockSpec(memory_space=pltpu.SMEM),
        ],
        out_specs=pl.BlockSpec((_BT, 1), lambda i: (i, 0)),
    )(xw, u4, qg, b.reshape(1, 1))
    return logits
